# baseline (speedup 1.0000x reference)
import functools

import jax
import jax.numpy as jnp
from jax import lax
from jax.experimental import pallas as pl
from jax.experimental.pallas import tpu as pltpu
from jax.experimental.pallas import tpu_sc as plsc

_SIZE = 8192
_EMB = 256
_NB = 64
_HW = 1024
_NC = 2
_NS = 16
_NW = _NC * _NS
_CPT = _EMB // _NW
_L = 16
_HALF = _NB // 2


def _transpose_body(x_ref, o_ref):
    o_ref[...] = x_ref[...].T


_transpose = pl.pallas_call(
    _transpose_body,
    grid=(2,),
    in_specs=[pl.BlockSpec((_SIZE, 128), lambda i: (0, i))],
    out_specs=pl.BlockSpec((128, _SIZE), lambda i: (i, 0)),
    out_shape=jax.ShapeDtypeStruct((_EMB, _SIZE), jnp.float32),
)


@functools.partial(
    pl.kernel,
    out_type=jax.ShapeDtypeStruct((_NB * _EMB * _HW,), jnp.float32),
    mesh=plsc.VectorSubcoreMesh(core_axis_name="c", subcore_axis_name="s"),
    compiler_params=pltpu.CompilerParams(needs_layout_passes=False),
    scratch_types=[
        pltpu.VMEM((_CPT * _SIZE,), jnp.float32),
        pltpu.VMEM((_HALF * _HW,), jnp.int32),
        pltpu.VMEM((_CPT * _HW,), jnp.float32),
    ],
)
def _codebook(idx_hbm, tblt_hbm, out_hbm, tbl_v, idx_v, out_v):
    wid = lax.axis_index("s") * _NC + lax.axis_index("c")
    c0 = wid * _CPT

    pltpu.sync_copy(tblt_hbm.at[pl.ds(c0 * _SIZE, _CPT * _SIZE)], tbl_v)

    for h in range(2):
        pltpu.sync_copy(idx_hbm.at[pl.ds(h * _HALF * _HW, _HALF * _HW)], idx_v)

        def batch_body(bl, _):
            def group_body(g, _):
                i16 = idx_v[pl.ds(bl * _HW + g * _L, _L)]
                for c in range(_CPT):
                    out_v[pl.ds(c * _HW + g * _L, _L)] = plsc.load_gather(
                        tbl_v, [i16 + (c * _SIZE)] if c else [i16]
                    )
                return 0

            lax.fori_loop(0, _HW // _L, group_body, 0)
            b = h * _HALF + bl
            out_off = (b * _EMB + c0) * _HW
            pltpu.sync_copy(out_v, out_hbm.at[pl.ds(out_off, _CPT * _HW)])
            return 0

        lax.fori_loop(0, _HALF, batch_body, 0)


def kernel(indices, table):
    idx_flat = indices.reshape(-1).astype(jnp.int32)
    tbl_t = _transpose(table).reshape(-1)
    out = _codebook(idx_flat, tbl_t)
    return out.reshape(_NB, _EMB, 32, 32)


# speedup vs baseline: 6.7886x; 6.7886x over previous
import functools

import jax
import jax.numpy as jnp
from jax import lax
from jax.experimental import pallas as pl
from jax.experimental.pallas import tpu as pltpu
from jax.experimental.pallas import tpu_sc as plsc

_SIZE = 8192
_EMB = 256
_NB = 64
_N = 65536
_NC = 2
_NS = 16
_NW = _NC * _NS
_P = _N // _NW
_C = 128


@functools.partial(
    pl.kernel,
    out_type=jax.ShapeDtypeStruct((_N, _EMB), jnp.float32),
    mesh=plsc.VectorSubcoreMesh(core_axis_name="c", subcore_axis_name="s"),
    compiler_params=pltpu.CompilerParams(needs_layout_passes=False),
    scratch_types=[
        pltpu.VMEM((_P,), jnp.int32),
        pltpu.VMEM((_C, _EMB), jnp.float32),
        pltpu.VMEM((_C, _EMB), jnp.float32),
        pltpu.SemaphoreType.DMA,
        pltpu.SemaphoreType.DMA,
    ],
)
def _gather(idx_hbm, tbl_hbm, out_hbm, idx_v, rows_a, rows_b, sem_a, sem_b):
    wid = lax.axis_index("s") * _NC + lax.axis_index("c")
    base = wid * _P
    pltpu.sync_copy(idx_hbm.at[pl.ds(base, _P)], idx_v)

    bufs = (rows_a, rows_b)
    sems = (sem_a, sem_b)
    nchunk = _P // _C

    pltpu.async_copy(tbl_hbm.at[idx_v.at[pl.ds(0, _C)]], rows_a, sem_a)

    def chunk_body(k, _):
        def do(parity):
            buf, sem = bufs[parity], sems[parity]
            obuf, osem = bufs[1 - parity], sems[1 - parity]
            pltpu.make_async_copy(
                tbl_hbm.at[idx_v.at[pl.ds(0, _C)]], buf, sem
            ).wait()
            @pl.when(k + 1 < nchunk)
            def _():
                pltpu.async_copy(
                    tbl_hbm.at[idx_v.at[pl.ds((k + 1) * _C, _C)]], obuf, osem
                )
            pltpu.sync_copy(buf, out_hbm.at[pl.ds(base + k * _C, _C), :])
            return 0

        lax.cond(k % 2 == 0, lambda: do(0), lambda: do(1))
        return 0

    lax.fori_loop(0, nchunk, chunk_body, 0)


def kernel(indices, table):
    idx_flat = indices.reshape(-1).astype(jnp.int32)
    g = _gather(idx_flat, table)
    x_q = g.reshape(_NB, 32, 32, _EMB)
    return jnp.moveaxis(x_q, -1, -3)


# speedup vs baseline: 6.9054x; 1.0172x over previous
import functools

import jax
import jax.numpy as jnp
from jax import lax
from jax.experimental import pallas as pl
from jax.experimental.pallas import tpu as pltpu
from jax.experimental.pallas import tpu_sc as plsc

_SIZE = 8192
_EMB = 256
_NB = 64
_N = 65536
_NC = 2
_NS = 16
_NW = _NC * _NS
_P = _N // _NW
_C = 128


_NBUF = 3
_NCHUNK = _P // _C


@functools.partial(
    pl.kernel,
    out_type=jax.ShapeDtypeStruct((_N, _EMB), jnp.float32),
    mesh=plsc.VectorSubcoreMesh(core_axis_name="c", subcore_axis_name="s"),
    compiler_params=pltpu.CompilerParams(needs_layout_passes=False),
    scratch_types=[
        pltpu.VMEM((_P,), jnp.int32),
        *[pltpu.VMEM((_C, _EMB), jnp.float32)] * _NBUF,
        *[pltpu.SemaphoreType.DMA] * _NBUF,
        *[pltpu.SemaphoreType.DMA] * _NBUF,
    ],
)
def _gather(idx_hbm, tbl_hbm, out_hbm, idx_v, *bs):
    bufs, gsems, wsems = bs[:_NBUF], bs[_NBUF:2 * _NBUF], bs[2 * _NBUF:]
    wid = lax.axis_index("s") * _NC + lax.axis_index("c")
    base = wid * _P
    pltpu.sync_copy(idx_hbm.at[pl.ds(base, _P)], idx_v)

    def start_gather(k):
        p = k % _NBUF
        pltpu.async_copy(
            tbl_hbm.at[idx_v.at[pl.ds(k * _C, _C)]], bufs[p], gsems[p]
        )

    def wait_gather(k):
        p = k % _NBUF
        pltpu.make_async_copy(
            tbl_hbm.at[idx_v.at[pl.ds(0, _C)]], bufs[p], gsems[p]
        ).wait()

    def start_write(k):
        p = k % _NBUF
        pltpu.async_copy(
            bufs[p], out_hbm.at[pl.ds(base + k * _C, _C), :], wsems[p]
        )

    def wait_write(k):
        p = k % _NBUF
        pltpu.make_async_copy(
            bufs[p], out_hbm.at[pl.ds(base, _C), :], wsems[p]
        ).wait()

    for k in range(min(_NBUF - 1, _NCHUNK)):
        start_gather(k)
    for k in range(_NCHUNK):
        wait_gather(k)
        if k + _NBUF - 1 < _NCHUNK:
            if k >= 1:
                wait_write(k - 1)
            start_gather(k + _NBUF - 1)
        start_write(k)
    for k in range(max(0, _NCHUNK - _NBUF), _NCHUNK):
        wait_write(k)


def kernel(indices, table):
    idx_flat = indices.reshape(-1).astype(jnp.int32)
    g = _gather(idx_flat, table)
    x_q = g.reshape(_NB, 32, 32, _EMB)
    return jnp.moveaxis(x_q, -1, -3)


# speedup vs baseline: 6.9268x; 1.0031x over previous
import functools

import jax
import jax.numpy as jnp
from jax import lax
from jax.experimental import pallas as pl
from jax.experimental.pallas import tpu as pltpu
from jax.experimental.pallas import tpu_sc as plsc

_SIZE = 8192
_EMB = 256
_NB = 64
_N = 65536
_NC = 2
_NS = 16
_NW = _NC * _NS
_P = _N // _NW
_C = 64


_NBUF = 6
_NCHUNK = _P // _C


@functools.partial(
    pl.kernel,
    out_type=jax.ShapeDtypeStruct((_N, _EMB), jnp.float32),
    mesh=plsc.VectorSubcoreMesh(core_axis_name="c", subcore_axis_name="s"),
    compiler_params=pltpu.CompilerParams(needs_layout_passes=False),
    scratch_types=[
        pltpu.VMEM((_P,), jnp.int32),
        *[pltpu.VMEM((_C, _EMB), jnp.float32)] * _NBUF,
        *[pltpu.SemaphoreType.DMA] * _NBUF,
        *[pltpu.SemaphoreType.DMA] * _NBUF,
    ],
)
def _gather(idx_hbm, tbl_hbm, out_hbm, idx_v, *bs):
    bufs, gsems, wsems = bs[:_NBUF], bs[_NBUF:2 * _NBUF], bs[2 * _NBUF:]
    wid = lax.axis_index("s") * _NC + lax.axis_index("c")
    base = wid * _P
    pltpu.sync_copy(idx_hbm.at[pl.ds(base, _P)], idx_v)

    def start_gather(k):
        p = k % _NBUF
        pltpu.async_copy(
            tbl_hbm.at[idx_v.at[pl.ds(k * _C, _C)]], bufs[p], gsems[p]
        )

    def wait_gather(k):
        p = k % _NBUF
        pltpu.make_async_copy(
            tbl_hbm.at[idx_v.at[pl.ds(0, _C)]], bufs[p], gsems[p]
        ).wait()

    def start_write(k):
        p = k % _NBUF
        pltpu.async_copy(
            bufs[p], out_hbm.at[pl.ds(base + k * _C, _C), :], wsems[p]
        )

    def wait_write(k):
        p = k % _NBUF
        pltpu.make_async_copy(
            bufs[p], out_hbm.at[pl.ds(base, _C), :], wsems[p]
        ).wait()

    for k in range(min(_NBUF - 1, _NCHUNK)):
        start_gather(k)
    for k in range(_NCHUNK):
        wait_gather(k)
        if k + _NBUF - 1 < _NCHUNK:
            if k >= 1:
                wait_write(k - 1)
            start_gather(k + _NBUF - 1)
        start_write(k)
    for k in range(max(0, _NCHUNK - _NBUF), _NCHUNK):
        wait_write(k)


def kernel(indices, table):
    idx_flat = indices.reshape(-1).astype(jnp.int32)
    g = _gather(idx_flat, table)
    x_q = g.reshape(_NB, 32, 32, _EMB)
    return jnp.moveaxis(x_q, -1, -3)
